# speedup vs baseline: 1.8270x; 1.8270x over previous
import functools

import jax
import jax.numpy as jnp
from jax import lax
from jax.experimental import pallas as pl
from jax.experimental.pallas import tpu as pltpu
from jax.experimental.pallas import tpu_sc as plsc

V = 8192
NTOK = 8192
NW = 32
ROWS_PER_W = NTOK // NW
C = 8
NCHUNK = ROWS_PER_W // C

_mesh = plsc.VectorSubcoreMesh(core_axis_name="c", subcore_axis_name="s")


@functools.partial(
    pl.kernel,
    mesh=_mesh,
    out_type=jax.ShapeDtypeStruct((NTOK, V), jnp.float32),
    scratch_types=[
        pltpu.VMEM((NCHUNK, C), jnp.int32),
        pltpu.VMEM((C, V), jnp.float32),
        pltpu.SemaphoreType.DMA,
    ],
)
def _gather_kernel(ids_hbm, table_hbm, out_hbm, idx_v, buf, sem):
    wid = lax.axis_index("s") * 2 + lax.axis_index("c")
    base = wid * ROWS_PER_W
    pltpu.sync_copy(ids_hbm.at[wid], idx_v)

    def body(i, carry):
        pltpu.async_copy(table_hbm.at[idx_v.at[i]], buf, sem).wait()
        pltpu.sync_copy(buf, out_hbm.at[pl.ds(base + i * C, C)])
        return carry

    lax.fori_loop(0, NCHUNK, body, 0)


def kernel(ids, table):
    B, T = ids.shape
    ids3 = ids.reshape(NW, NCHUNK, C).astype(jnp.int32)
    out = _gather_kernel(ids3, table)
    return out.reshape(B, T, V)


# speedup vs baseline: 1.9314x; 1.0572x over previous
import functools

import jax
import jax.numpy as jnp
from jax import lax
from jax.experimental import pallas as pl
from jax.experimental.pallas import tpu as pltpu
from jax.experimental.pallas import tpu_sc as plsc

V = 8192
NTOK = 8192
NW = 32
ROWS_PER_W = NTOK // NW
C = 2
NCHUNK = ROWS_PER_W // C
NBUF = 4

_mesh = plsc.VectorSubcoreMesh(core_axis_name="c", subcore_axis_name="s")


@functools.partial(
    pl.kernel,
    mesh=_mesh,
    out_type=jax.ShapeDtypeStruct((NTOK, V), jnp.float32),
    scratch_types=[
        pltpu.VMEM((NCHUNK, C), jnp.int32),
    ]
    + [pltpu.VMEM((C, V), jnp.float32) for _ in range(NBUF)]
    + [pltpu.SemaphoreType.DMA for _ in range(2 * NBUF)],
)
def _gather_kernel(ids_hbm, table_hbm, out_hbm, idx_v, *bufs_and_sems):
    bufs = bufs_and_sems[:NBUF]
    gsem = bufs_and_sems[NBUF:2 * NBUF]
    wsem = bufs_and_sems[2 * NBUF:]

    wid = lax.axis_index("s") * 2 + lax.axis_index("c")
    base = wid * ROWS_PER_W
    pltpu.sync_copy(ids_hbm.at[wid], idx_v)

    def start_gather(chunk, b):
        pltpu.make_async_copy(
            table_hbm.at[idx_v.at[chunk]], bufs[b], gsem[b]
        ).start()

    def drain_gather(b):
        pltpu.make_async_copy(
            table_hbm.at[pl.ds(0, C)], bufs[b], gsem[b]
        ).wait()

    def start_write(chunk, b):
        pltpu.make_async_copy(
            bufs[b], out_hbm.at[pl.ds(base + chunk * C, C)], wsem[b]
        ).start()

    def drain_write(b):
        pltpu.make_async_copy(
            bufs[b], out_hbm.at[pl.ds(base, C)], wsem[b]
        ).wait()

    for b in range(NBUF):
        start_gather(b, b)

    def outer(it, carry):
        g = it * NBUF
        for b in range(NBUF):
            drain_gather(b)
            start_write(g + b, b)
        for b in range(NBUF):
            drain_write(b)
            start_gather(g + NBUF + b, b)
        return carry

    lax.fori_loop(0, NCHUNK // NBUF - 1, outer, 0)

    g = NCHUNK - NBUF
    for b in range(NBUF):
        drain_gather(b)
        start_write(g + b, b)
    for b in range(NBUF):
        drain_write(b)


def kernel(ids, table):
    B, T = ids.shape
    ids3 = ids.reshape(NW, NCHUNK, C).astype(jnp.int32)
    out = _gather_kernel(ids3, table)
    return out.reshape(B, T, V)


# speedup vs baseline: 1.9397x; 1.0042x over previous
import functools

import jax
import jax.numpy as jnp
from jax import lax
from jax.experimental import pallas as pl
from jax.experimental.pallas import tpu as pltpu
from jax.experimental.pallas import tpu_sc as plsc

V = 8192
NTOK = 8192
NW = 32
ROWS_PER_W = NTOK // NW
C = 2
NCHUNK = ROWS_PER_W // C
NBUF = 4
NG = NCHUNK // NBUF

_mesh = plsc.VectorSubcoreMesh(core_axis_name="c", subcore_axis_name="s")


@functools.partial(
    pl.kernel,
    mesh=_mesh,
    out_type=jax.ShapeDtypeStruct((NTOK, V), jnp.float32),
    scratch_types=[
        pltpu.VMEM((NCHUNK, C), jnp.int32),
    ]
    + [pltpu.VMEM((C, V), jnp.float32) for _ in range(NBUF)]
    + [pltpu.SemaphoreType.DMA for _ in range(2 * NBUF)],
)
def _gather_kernel(ids_hbm, table_hbm, out_hbm, idx_v, *bufs_and_sems):
    bufs = bufs_and_sems[:NBUF]
    gsem = bufs_and_sems[NBUF:2 * NBUF]
    wsem = bufs_and_sems[2 * NBUF:]

    wid = lax.axis_index("s") * 2 + lax.axis_index("c")
    base = wid * ROWS_PER_W
    pltpu.sync_copy(ids_hbm.at[wid], idx_v)

    def start_gather(chunk, b):
        pltpu.make_async_copy(
            table_hbm.at[idx_v.at[chunk]], bufs[b], gsem[b]
        ).start()

    def drain_gather(b):
        pltpu.make_async_copy(
            table_hbm.at[pl.ds(0, C)], bufs[b], gsem[b]
        ).wait()

    def start_write(chunk, b):
        pltpu.make_async_copy(
            bufs[b], out_hbm.at[pl.ds(base + chunk * C, C)], wsem[b]
        ).start()

    def drain_write(b):
        pltpu.make_async_copy(
            bufs[b], out_hbm.at[pl.ds(base, C)], wsem[b]
        ).wait()

    start_gather(0, 0)
    start_gather(1, 1)
    start_gather(2, 2)
    drain_gather(0)
    start_write(0, 0)
    start_gather(3, 3)
    drain_gather(1)
    start_write(1, 1)

    def outer(g, carry):
        for b in range(NBUF):
            i = g * NBUF + b
            drain_write(b)
            start_gather(i, b)
            b2 = (b - 2) % NBUF
            drain_gather(b2)
            start_write(i - 2, b2)
        return carry

    lax.fori_loop(1, NG, outer, 0)

    drain_gather(2)
    start_write(NCHUNK - 2, 2)
    drain_gather(3)
    start_write(NCHUNK - 1, 3)
    for b in range(NBUF):
        drain_write(b)


def kernel(ids, table):
    B, T = ids.shape
    ids3 = ids.reshape(NW, NCHUNK, C).astype(jnp.int32)
    out = _gather_kernel(ids3, table)
    return out.reshape(B, T, V)
